# baseline (speedup 1.0000x reference)
import functools

import jax
import jax.numpy as jnp
from jax import lax
from jax.experimental import pallas as pl
from jax.experimental.pallas import tpu as pltpu
from jax.experimental.pallas import tpu_sc as plsc

N = 10000
D = 128
E = 160000
ROWS = 2000

NW = 16
EPW = E // NW
W = 40
NWIN = EPW // W
NPAD = 10240
RPT = NPAD // 16


def _full(shape):
    return pl.BlockSpec(shape, lambda i: (0,) * len(shape))


def _rows(width):
    return pl.BlockSpec((ROWS, width), lambda i: (i, 0))


def _linrelu_body(x_ref, w_ref, b_ref, o_ref):
    o_ref[...] = jnp.maximum(
        jnp.dot(x_ref[...], w_ref[...], preferred_element_type=jnp.float32)
        + b_ref[...], 0.0)


def _linrelu(x, w, b):
    return pl.pallas_call(
        _linrelu_body,
        grid=(N // ROWS,),
        in_specs=[_rows(D), _full((D, D)), _full((1, D))],
        out_specs=_rows(D),
        out_shape=jax.ShapeDtypeStruct((N, D), jnp.float32),
    )(x, w, b.reshape(1, D))


def _mlp2_body(x_ref, w1_ref, b1_ref, w2_ref, b2_ref, o_ref):
    h = jnp.maximum(
        jnp.dot(x_ref[...], w1_ref[...], preferred_element_type=jnp.float32)
        + b1_ref[...], 0.0)
    o_ref[...] = (jnp.dot(h, w2_ref[...], preferred_element_type=jnp.float32)
                  + b2_ref[...])


def _mlp2(x, w1, b1, w2, b2):
    return pl.pallas_call(
        _mlp2_body,
        grid=(N // ROWS,),
        in_specs=[_rows(D), _full((D, D)), _full((1, D)),
                  _full((D, D)), _full((1, D))],
        out_specs=_rows(D),
        out_shape=jax.ShapeDtypeStruct((N, D), jnp.float32),
    )(x, w1, b1.reshape(1, D), w2, b2.reshape(1, D))


_DNUMS = lax.GatherDimensionNumbers(
    offset_dims=(), collapsed_slice_dims=(0,), start_index_map=(0,))


def _permute(v, idx):
    return lax.gather(v, idx.reshape(16, 1), _DNUMS, (1,),
                      mode=lax.GatherScatterMode.PROMISE_IN_BOUNDS)


def _hsum(v, bf_idx):
    for idx in bf_idx:
        v = v + _permute(v, idx)
    return v


def _rsqrt_vec(x):
    i = lax.bitcast_convert_type(x, jnp.int32)
    i = jnp.int32(0x5F3759DF) - lax.shift_right_arithmetic(i, 1)
    y = lax.bitcast_convert_type(i, jnp.float32)
    half = 0.5 * x
    for _ in range(2):
        y = y * (1.5 - half * y * y)
    return y


def _bf16_bits(v):
    i = lax.bitcast_convert_type(v, jnp.int32)
    return lax.shift_right_logical(
        i + 0x7FFF + (lax.shift_right_logical(i, 16) & 1), 16)


def _edge_body(hsrc_hbm, pred_hbm, src_hbm, dst_hbm, aggr_out,
               is_b, id_b, ids_b, xb, pb, ob, sh_aggr,
               six, sxx, spx, ssx):
    if True:
        sid = lax.axis_index("s")
        wid = sid
        zero16 = jnp.zeros((16,), jnp.float32)

        def _zo(e, c):
            for k in range(8):
                ob[0][e, pl.ds(k * 16, 16)] = zero16
            return c
        lax.fori_loop(0, W, _zo, 0)

        r0 = sid * RPT
        for j in range(RPT // W):
            pltpu.sync_copy(ob[0], sh_aggr.at[pl.ds(r0 + j * W, W)])
        plsc.subcore_barrier()

        bf_idx = [lax.iota(jnp.int32, 16) ^ sh for sh in (8, 4, 2, 1)]

        def idx_start(w, b):
            base = wid * EPW + w * W
            pltpu.async_copy(src_hbm.at[pl.ds(base, W)], is_b[b], six[b])
            pltpu.async_copy(dst_hbm.at[pl.ds(base, W)], id_b[b], six[b])

        def idx_wait(b):
            pltpu.make_async_copy(src_hbm.at[pl.ds(0, W)], is_b[b],
                                  six[b]).wait()
            pltpu.make_async_copy(dst_hbm.at[pl.ds(0, W)], id_b[b],
                                  six[b]).wait()

        def gather_start(b):
            pltpu.async_copy(hsrc_hbm.at[is_b[b]], xb[b], sxx[b])
            pltpu.async_copy(pred_hbm.at[id_b[b]], pb[b], spx[b])

        def gather_wait(b):
            pltpu.make_async_copy(hsrc_hbm.at[is_b[b]], xb[b], sxx[b]).wait()
            pltpu.make_async_copy(pred_hbm.at[id_b[b]], pb[b], spx[b]).wait()

        def _edge_loop(b):
            def _edge(e):
                d = []
                for k in range(8):
                    xk = xb[b][e, pl.ds(k * 16, 16)]
                    pk = pb[b][e, pl.ds(k * 16, 16)]
                    d.append(xk - pk)
                s1 = d[0]
                for k in range(1, 8):
                    s1 = s1 + d[k]
                s2 = d[0] * d[0]
                for k in range(1, 8):
                    s2 = s2 + d[k] * d[k]
                t1 = _hsum(s1, bf_idx)
                t2 = _hsum(s2, bf_idx)
                mu = t1 * (1.0 / 128.0)
                var = t2 * (1.0 / 128.0) - mu * mu
                w = _rsqrt_vec(var + 1e-5)
                mw = mu * w
                for k in range(8):
                    ob[b][e, pl.ds(k * 16, 16)] = d[k] * w - mw

            plsc.parallel_loop(0, W, 1, unroll=4)(_edge)

        def step(w, b):
            idx_wait(b ^ 1)
            gather_start(b ^ 1)
            gather_wait(b)
            idx_start(jnp.minimum(w + 2, NWIN - 1), b)

            @pl.when(w >= 2)
            def _():
                pltpu.make_async_copy(ob[b], sh_aggr.at[ids_b[b]],
                                      ssx[b]).wait()
            for off in (0, 16, W - 16):
                ids_b[b][pl.ds(off, 16)] = id_b[b][pl.ds(off, 16)]
            _edge_loop(b)
            pltpu.async_copy(ob[b], sh_aggr.at[ids_b[b]], ssx[b], add=True)

        idx_start(jnp.int32(0), 0)
        idx_start(jnp.int32(1), 1)
        idx_wait(0)
        gather_start(0)

        def _outer(i, c):
            step(2 * i, 0)
            step(2 * i + 1, 1)
            return c
        lax.fori_loop(0, NWIN // 2, _outer, 0)
        idx_wait(1)
        gather_wait(0)
        pltpu.make_async_copy(ob[0], sh_aggr.at[ids_b[0]], ssx[0]).wait()
        pltpu.make_async_copy(ob[1], sh_aggr.at[ids_b[1]], ssx[1]).wait()
        plsc.subcore_barrier()

        def _mk_cvt(which, obase):
            def _cvt(rp, c):
                for h in range(2):
                    for p in range(4):
                        u = which[2 * rp + h, pl.ds(32 * p, 16)]
                        v = which[2 * rp + h, pl.ds(32 * p + 16, 16)]
                        word = ((lax.shift_left(_bf16_bits(v), 16))
                                | _bf16_bits(u))
                        pb[0][obase + rp, pl.ds(64 * h + 16 * p, 16)] = (
                            lax.bitcast_convert_type(word, jnp.float32))
                return c
            return _cvt

        for j2 in range(RPT // (2 * W)):
            pltpu.sync_copy(sh_aggr.at[pl.ds(r0 + 2 * j2 * W, W)], xb[0])
            pltpu.sync_copy(sh_aggr.at[pl.ds(r0 + (2 * j2 + 1) * W, W)],
                            xb[1])
            lax.fori_loop(0, W // 2, _mk_cvt(xb[0], 0), 0)
            lax.fori_loop(0, W // 2, _mk_cvt(xb[1], W // 2), 0)
            pltpu.sync_copy(
                pb[0].at[pl.ds(0, W)],
                aggr_out.at[pl.ds(sid * (RPT // 2) + j2 * W, W)])


@functools.partial(
    pl.kernel,
    mesh=plsc.VectorSubcoreMesh(core_axis_name="c", subcore_axis_name="s",
                                num_cores=1),
    out_type=[jax.ShapeDtypeStruct((NPAD // 2, D), jnp.float32)],
    scratch_types=[
        [pltpu.VMEM((W,), jnp.int32)] * 2,
        [pltpu.VMEM((W,), jnp.int32)] * 2,
        [pltpu.VMEM((W,), jnp.int32)] * 2,
        [pltpu.VMEM((W, D), jnp.float32)] * 2,
        [pltpu.VMEM((W, D), jnp.float32)] * 2,
        [pltpu.VMEM((W, D), jnp.float32)] * 2,
        pltpu.VMEM_SHARED((NPAD, D), jnp.float32),
        [pltpu.SemaphoreType.DMA] * 2,
        [pltpu.SemaphoreType.DMA] * 2,
        [pltpu.SemaphoreType.DMA] * 2,
        [pltpu.SemaphoreType.DMA] * 2,
    ],
)
def _edge_sc(hsrc_hbm, pred_hbm, src_hbm, dst_hbm, aggr_out,
             is_b, id_b, ids_b, xb, pb, ob, sh_aggr,
             six, sxx, spx, ssx):
    _edge_body(hsrc_hbm, pred_hbm, src_hbm, dst_hbm, aggr_out,
               is_b, id_b, ids_b, xb, pb, ob, sh_aggr,
               six, sxx, spx, ssx)


def _unpack_aggr(aggr_raw):
    bf = lax.bitcast_convert_type(aggr_raw, jnp.bfloat16)
    return (bf.reshape(NPAD, 4, 16, 2).transpose(0, 1, 3, 2)
            .reshape(NPAD, D))


WC = 80
NWINC = EPW // WC


def _cnt_body(dst_hbm, cnt_out, idx_d, ones_b, cw, sh_cnt, sic, ssc):
    if True:
        sid = lax.axis_index("s")
        zero16 = jnp.zeros((16,), jnp.float32)
        one16 = jnp.ones((16,), jnp.float32)

        def _zo(e, c):
            for k in range(8):
                ones_b[e, pl.ds(k * 16, 16)] = zero16
            return c
        lax.fori_loop(0, WC, _zo, 0)

        r0 = sid * RPT
        for j in range(RPT // WC):
            pltpu.sync_copy(ones_b, sh_cnt.at[pl.ds(r0 + j * WC, WC)])

        def _oi(e, c):
            ones_b[e, pl.ds(0, 16)] = one16
            return c
        lax.fori_loop(0, WC, _oi, 0)
        plsc.subcore_barrier()

        def idx_start(w, b):
            pltpu.async_copy(dst_hbm.at[pl.ds(sid * EPW + w * WC, WC)],
                             idx_d[b], sic[b])

        def idx_wait(b):
            pltpu.make_async_copy(dst_hbm.at[pl.ds(0, WC)], idx_d[b],
                                  sic[b]).wait()

        idx_start(jnp.int32(0), 0)

        def step(w, b):
            idx_wait(b)
            pltpu.async_copy(ones_b, sh_cnt.at[idx_d[b]], ssc[b], add=True)

            @pl.when(w >= 1)
            def _():
                pltpu.make_async_copy(ones_b, sh_cnt.at[idx_d[b ^ 1]],
                                      ssc[b ^ 1]).wait()
            idx_start(jnp.minimum(w + 1, NWINC - 1), b ^ 1)

        def _outer(i, c):
            step(2 * i, 0)
            step(2 * i + 1, 1)
            return c
        lax.fori_loop(0, NWINC // 2, _outer, 0)
        step(jnp.int32(NWINC - 1), 0)
        idx_wait(1)
        pltpu.make_async_copy(ones_b, sh_cnt.at[idx_d[0]], ssc[0]).wait()
        plsc.subcore_barrier()

        iota_f = lax.iota(jnp.int32, 16).astype(jnp.float32)
        onehots = [jnp.maximum(1.0 - jnp.abs(iota_f - float(r)), 0.0)
                   for r in range(16)]
        for j in range(RPT // WC):
            pltpu.sync_copy(sh_cnt.at[pl.ds(r0 + j * WC, WC)], ones_b)
            for q in range(WC // 16):
                acc = ones_b[q * 16, pl.ds(0, 16)] * onehots[0]
                for r in range(1, 16):
                    acc = acc + ones_b[q * 16 + r, pl.ds(0, 16)] * onehots[r]
                cw[j * (WC // 16) + q, :] = acc
        pltpu.sync_copy(cw, cnt_out.at[pl.ds(sid * (RPT // 16), RPT // 16)])


@functools.partial(
    pl.kernel,
    mesh=plsc.VectorSubcoreMesh(core_axis_name="c", subcore_axis_name="s",
                                num_cores=1),
    out_type=[jax.ShapeDtypeStruct((NPAD // 16, 16), jnp.float32)],
    scratch_types=[
        [pltpu.VMEM((WC,), jnp.int32)] * 2,
        pltpu.VMEM((WC, D), jnp.float32),
        pltpu.VMEM((RPT // 16, 16), jnp.float32),
        pltpu.VMEM_SHARED((NPAD, D), jnp.float32),
        [pltpu.SemaphoreType.DMA] * 2,
        [pltpu.SemaphoreType.DMA] * 2,
    ],
)
def _cnt_sc(dst_hbm, cnt_out, idx_d, ones_b, cw, sh_cnt, sic, ssc):
    _cnt_body(dst_hbm, cnt_out, idx_d, ones_b, cw, sh_cnt, sic, ssc)


def _update_body(h_ref, a0_ref, c0_ref, wt_ref, wb_ref,
                 bu_ref, lg_ref, lb_ref, g_ref, b_ref, o_ref):
    h = h_ref[...]
    cnt = c0_ref[...]
    s = a0_ref[...].astype(jnp.float32)
    am = (lg_ref[...] * s + lb_ref[...] * cnt) / jnp.clip(cnt, 1.0, None)
    u = (jnp.dot(h, wt_ref[...], preferred_element_type=jnp.float32)
         + jnp.dot(am, wb_ref[...], preferred_element_type=jnp.float32)
         + bu_ref[...])
    mu = jnp.mean(u, axis=-1, keepdims=True)
    var = jnp.mean(u * u, axis=-1, keepdims=True) - mu * mu
    y = (u - mu) * lax.rsqrt(var + 1e-5) * g_ref[...] + b_ref[...]
    o_ref[...] = jnp.maximum(y, 0.0) + h


def _update(h, aggr, cnt, p_conv, ln):
    return pl.pallas_call(
        _update_body,
        grid=(N // ROWS,),
        in_specs=[_rows(D),
                  pl.BlockSpec((ROWS, D), lambda i: (i, 0)),
                  pl.BlockSpec((ROWS, 1), lambda i: (i, 0)),
                  _full((D, D)), _full((D, D)), _full((1, D)),
                  _full((1, D)), _full((1, D)),
                  _full((1, D)), _full((1, D))],
        out_specs=_rows(D),
        out_shape=jax.ShapeDtypeStruct((N, D), jnp.float32),
    )(h, aggr, cnt,
      p_conv["upd"]["W"][:D], p_conv["upd"]["W"][D:],
      p_conv["upd"]["b"].reshape(1, D),
      p_conv["ln_g"].reshape(1, D), p_conv["ln_b"].reshape(1, D),
      ln["g"].reshape(1, D), ln["b"].reshape(1, D))


def _head_body(x_ref, w1_ref, b1_ref, w2_ref, b2_ref, o_ref):
    z = jnp.maximum(
        jnp.dot(x_ref[...], w1_ref[...], preferred_element_type=jnp.float32)
        + b1_ref[...], 0.0)
    o_ref[...] = (jnp.dot(z, w2_ref[...], preferred_element_type=jnp.float32)
                  + b2_ref[...])


def _head(x, p1, p2):
    return pl.pallas_call(
        _head_body,
        grid=(N // ROWS,),
        in_specs=[_rows(D), _full((D, D // 2)), _full((1, D // 2)),
                  _full((D // 2, 1)), _full((1, 1))],
        out_specs=pl.BlockSpec((ROWS, 1), lambda i: (i, 0)),
        out_shape=jax.ShapeDtypeStruct((N, 1), jnp.float32),
    )(x, p1["W"], p1["b"].reshape(1, -1), p2["W"], p2["b"].reshape(1, 1))


def kernel(x_drivers, x_races, params, ei_dr_ra, ei_ra_dr):
    ei_dr_ra = ei_dr_ra.astype(jnp.int32)
    ei_ra_dr = ei_ra_dr.astype(jnp.int32)
    enc = params["enc"]
    h_dr = _linrelu(x_drivers, enc["drivers"]["W"], enc["drivers"]["b"])
    h_ra = _linrelu(x_races, enc["races"]["W"], enc["races"]["b"])
    cnt_ra, = _cnt_sc(ei_dr_ra[1])
    cnt_dr, = _cnt_sc(ei_ra_dr[1])
    cnt_ra = cnt_ra.reshape(NPAD, 1)
    cnt_dr = cnt_dr.reshape(NPAD, 1)
    for lp in params["layers"]:
        p_ra = lp["dr_ra"]
        p_dr = lp["ra_dr"]
        pred_ra = _mlp2(h_ra, p_ra["pred1"]["W"], p_ra["pred1"]["b"],
                        p_ra["pred2"]["W"], p_ra["pred2"]["b"])
        pred_dr = _mlp2(h_dr, p_dr["pred1"]["W"], p_dr["pred1"]["b"],
                        p_dr["pred2"]["W"], p_dr["pred2"]["b"])
        aggr_ra, = _edge_sc(h_dr, pred_ra, ei_dr_ra[0], ei_dr_ra[1])
        aggr_dr, = _edge_sc(h_ra, pred_dr, ei_ra_dr[0], ei_ra_dr[1])
        aggr_ra = _unpack_aggr(aggr_ra)
        aggr_dr = _unpack_aggr(aggr_dr)
        new_ra = _update(h_ra, aggr_ra, cnt_ra, p_ra, lp["ln"]["races"])
        new_dr = _update(h_dr, aggr_dr, cnt_dr, p_dr, lp["ln"]["drivers"])
        h_ra, h_dr = new_ra, new_dr
    return _head(h_dr, params["head1"], params["head2"])


# speedup vs baseline: 1.0390x; 1.0390x over previous
import functools

import jax
import jax.numpy as jnp
from jax import lax
from jax.experimental import pallas as pl
from jax.experimental.pallas import tpu as pltpu
from jax.experimental.pallas import tpu_sc as plsc

N = 10000
D = 128
E = 160000
ROWS = 2000

NW = 16
EPW = E // NW
W = 40
NWIN = EPW // W
NPAD = 10240
RPT = NPAD // 16


def _full(shape):
    return pl.BlockSpec(shape, lambda i: (0,) * len(shape))


def _rows(width):
    return pl.BlockSpec((ROWS, width), lambda i: (i, 0))


def _linrelu_body(x_ref, w_ref, b_ref, o_ref):
    o_ref[...] = jnp.maximum(
        jnp.dot(x_ref[...], w_ref[...], preferred_element_type=jnp.float32)
        + b_ref[...], 0.0)


def _linrelu(x, w, b):
    return pl.pallas_call(
        _linrelu_body,
        grid=(N // ROWS,),
        in_specs=[_rows(D), _full((D, D)), _full((1, D))],
        out_specs=_rows(D),
        out_shape=jax.ShapeDtypeStruct((N, D), jnp.float32),
    )(x, w, b.reshape(1, D))


def _mlp2_body(x_ref, w1_ref, b1_ref, w2_ref, b2_ref, o_ref):
    h = jnp.maximum(
        jnp.dot(x_ref[...], w1_ref[...], preferred_element_type=jnp.float32)
        + b1_ref[...], 0.0)
    o_ref[...] = (jnp.dot(h, w2_ref[...], preferred_element_type=jnp.float32)
                  + b2_ref[...])


def _mlp2(x, w1, b1, w2, b2):
    return pl.pallas_call(
        _mlp2_body,
        grid=(N // ROWS,),
        in_specs=[_rows(D), _full((D, D)), _full((1, D)),
                  _full((D, D)), _full((1, D))],
        out_specs=_rows(D),
        out_shape=jax.ShapeDtypeStruct((N, D), jnp.float32),
    )(x, w1, b1.reshape(1, D), w2, b2.reshape(1, D))


_DNUMS = lax.GatherDimensionNumbers(
    offset_dims=(), collapsed_slice_dims=(0,), start_index_map=(0,))


def _permute(v, idx):
    return lax.gather(v, idx.reshape(16, 1), _DNUMS, (1,),
                      mode=lax.GatherScatterMode.PROMISE_IN_BOUNDS)


def _hsum(v, bf_idx):
    for idx in bf_idx:
        v = v + _permute(v, idx)
    return v


def _rsqrt_vec(x):
    i = lax.bitcast_convert_type(x, jnp.int32)
    i = jnp.int32(0x5F3759DF) - lax.shift_right_arithmetic(i, 1)
    y = lax.bitcast_convert_type(i, jnp.float32)
    half = 0.5 * x
    for _ in range(2):
        y = y * (1.5 - half * y * y)
    return y


def _bf16_bits(v):
    i = lax.bitcast_convert_type(v, jnp.int32)
    return lax.shift_right_logical(
        i + 0x7FFF + (lax.shift_right_logical(i, 16) & 1), 16)


def _edge_body(hsrc_hbm, pred_hbm, src_hbm, dst_hbm, aggr_out,
               is_b, id_b, ids_b, xb, pb, ob, sh_aggr,
               six, sxx, spx, ssx):
    if True:
        sid = lax.axis_index("s")
        wid = sid
        zero16 = jnp.zeros((16,), jnp.float32)

        def _zo(e, c):
            for k in range(8):
                ob[0][e, pl.ds(k * 16, 16)] = zero16
            return c
        lax.fori_loop(0, W, _zo, 0)

        r0 = sid * RPT
        for j in range(RPT // W):
            pltpu.sync_copy(ob[0], sh_aggr.at[pl.ds(r0 + j * W, W)])
        plsc.subcore_barrier()

        bf_idx = [lax.iota(jnp.int32, 16) ^ sh for sh in (8, 4, 2, 1)]

        def idx_start(w, b):
            base = wid * EPW + w * W
            pltpu.async_copy(src_hbm.at[pl.ds(base, W)], is_b[b], six[b])
            pltpu.async_copy(dst_hbm.at[pl.ds(base, W)], id_b[b], six[b])

        def idx_wait(b):
            pltpu.make_async_copy(src_hbm.at[pl.ds(0, W)], is_b[b],
                                  six[b]).wait()
            pltpu.make_async_copy(dst_hbm.at[pl.ds(0, W)], id_b[b],
                                  six[b]).wait()

        def gather_start(b):
            pltpu.async_copy(hsrc_hbm.at[is_b[b]], xb[b], sxx[b])
            pltpu.async_copy(pred_hbm.at[id_b[b]], pb[b], spx[b])

        def gather_wait(b):
            pltpu.make_async_copy(hsrc_hbm.at[is_b[b]], xb[b], sxx[b]).wait()
            pltpu.make_async_copy(pred_hbm.at[id_b[b]], pb[b], spx[b]).wait()

        def _edge_loop(b):
            def _edge(e):
                d = []
                for k in range(8):
                    xk = xb[b][e, pl.ds(k * 16, 16)]
                    pk = pb[b][e, pl.ds(k * 16, 16)]
                    d.append(xk - pk)
                s1 = d[0]
                for k in range(1, 8):
                    s1 = s1 + d[k]
                s2 = d[0] * d[0]
                for k in range(1, 8):
                    s2 = s2 + d[k] * d[k]
                t1 = _hsum(s1, bf_idx)
                t2 = _hsum(s2, bf_idx)
                mu = t1 * (1.0 / 128.0)
                var = t2 * (1.0 / 128.0) - mu * mu
                w = _rsqrt_vec(var + 1e-5)
                mw = mu * w
                for k in range(8):
                    ob[b][e, pl.ds(k * 16, 16)] = d[k] * w - mw

            plsc.parallel_loop(0, W, 1, unroll=2)(_edge)

        def step(w, b):
            idx_wait(b ^ 1)
            gather_start(b ^ 1)
            gather_wait(b)
            idx_start(jnp.minimum(w + 2, NWIN - 1), b)

            @pl.when(w >= 2)
            def _():
                pltpu.make_async_copy(ob[b], sh_aggr.at[ids_b[b]],
                                      ssx[b]).wait()
            for off in (0, 16, W - 16):
                ids_b[b][pl.ds(off, 16)] = id_b[b][pl.ds(off, 16)]
            _edge_loop(b)
            pltpu.async_copy(ob[b], sh_aggr.at[ids_b[b]], ssx[b], add=True)

        idx_start(jnp.int32(0), 0)
        idx_start(jnp.int32(1), 1)
        idx_wait(0)
        gather_start(0)

        def _outer(i, c):
            step(2 * i, 0)
            step(2 * i + 1, 1)
            return c
        lax.fori_loop(0, NWIN // 2, _outer, 0)
        idx_wait(1)
        gather_wait(0)
        pltpu.make_async_copy(ob[0], sh_aggr.at[ids_b[0]], ssx[0]).wait()
        pltpu.make_async_copy(ob[1], sh_aggr.at[ids_b[1]], ssx[1]).wait()
        plsc.subcore_barrier()

        def _mk_cvt(which, obase):
            def _cvt(rp, c):
                for h in range(2):
                    for p in range(4):
                        u = which[2 * rp + h, pl.ds(32 * p, 16)]
                        v = which[2 * rp + h, pl.ds(32 * p + 16, 16)]
                        word = ((lax.shift_left(_bf16_bits(v), 16))
                                | _bf16_bits(u))
                        pb[0][obase + rp, pl.ds(64 * h + 16 * p, 16)] = (
                            lax.bitcast_convert_type(word, jnp.float32))
                return c
            return _cvt

        for j2 in range(RPT // (2 * W)):
            pltpu.sync_copy(sh_aggr.at[pl.ds(r0 + 2 * j2 * W, W)], xb[0])
            pltpu.sync_copy(sh_aggr.at[pl.ds(r0 + (2 * j2 + 1) * W, W)],
                            xb[1])
            lax.fori_loop(0, W // 2, _mk_cvt(xb[0], 0), 0)
            lax.fori_loop(0, W // 2, _mk_cvt(xb[1], W // 2), 0)
            pltpu.sync_copy(
                pb[0].at[pl.ds(0, W)],
                aggr_out.at[pl.ds(sid * (RPT // 2) + j2 * W, W)])


@functools.partial(
    pl.kernel,
    mesh=plsc.VectorSubcoreMesh(core_axis_name="c", subcore_axis_name="s",
                                num_cores=1),
    out_type=[jax.ShapeDtypeStruct((NPAD // 2, D), jnp.float32)],
    scratch_types=[
        [pltpu.VMEM((W,), jnp.int32)] * 2,
        [pltpu.VMEM((W,), jnp.int32)] * 2,
        [pltpu.VMEM((W,), jnp.int32)] * 2,
        [pltpu.VMEM((W, D), jnp.float32)] * 2,
        [pltpu.VMEM((W, D), jnp.float32)] * 2,
        [pltpu.VMEM((W, D), jnp.float32)] * 2,
        pltpu.VMEM_SHARED((NPAD, D), jnp.float32),
        [pltpu.SemaphoreType.DMA] * 2,
        [pltpu.SemaphoreType.DMA] * 2,
        [pltpu.SemaphoreType.DMA] * 2,
        [pltpu.SemaphoreType.DMA] * 2,
    ],
)
def _edge_sc(hsrc_hbm, pred_hbm, src_hbm, dst_hbm, aggr_out,
             is_b, id_b, ids_b, xb, pb, ob, sh_aggr,
             six, sxx, spx, ssx):
    _edge_body(hsrc_hbm, pred_hbm, src_hbm, dst_hbm, aggr_out,
               is_b, id_b, ids_b, xb, pb, ob, sh_aggr,
               six, sxx, spx, ssx)


def _unpack_aggr(aggr_raw):
    bf = lax.bitcast_convert_type(aggr_raw, jnp.bfloat16)
    return (bf.reshape(NPAD, 4, 16, 2).transpose(0, 1, 3, 2)
            .reshape(NPAD, D))


WC = 80
NWINC = EPW // WC


def _cnt_body(dst_hbm, cnt_out, idx_d, ones_b, cw, sh_cnt, sic, ssc):
    if True:
        sid = lax.axis_index("s")
        zero16 = jnp.zeros((16,), jnp.float32)
        one16 = jnp.ones((16,), jnp.float32)

        def _zo(e, c):
            for k in range(8):
                ones_b[e, pl.ds(k * 16, 16)] = zero16
            return c
        lax.fori_loop(0, WC, _zo, 0)

        r0 = sid * RPT
        for j in range(RPT // WC):
            pltpu.sync_copy(ones_b, sh_cnt.at[pl.ds(r0 + j * WC, WC)])

        def _oi(e, c):
            ones_b[e, pl.ds(0, 16)] = one16
            return c
        lax.fori_loop(0, WC, _oi, 0)
        plsc.subcore_barrier()

        def idx_start(w, b):
            pltpu.async_copy(dst_hbm.at[pl.ds(sid * EPW + w * WC, WC)],
                             idx_d[b], sic[b])

        def idx_wait(b):
            pltpu.make_async_copy(dst_hbm.at[pl.ds(0, WC)], idx_d[b],
                                  sic[b]).wait()

        idx_start(jnp.int32(0), 0)

        def step(w, b):
            idx_wait(b)
            pltpu.async_copy(ones_b, sh_cnt.at[idx_d[b]], ssc[b], add=True)

            @pl.when(w >= 1)
            def _():
                pltpu.make_async_copy(ones_b, sh_cnt.at[idx_d[b ^ 1]],
                                      ssc[b ^ 1]).wait()
            idx_start(jnp.minimum(w + 1, NWINC - 1), b ^ 1)

        def _outer(i, c):
            step(2 * i, 0)
            step(2 * i + 1, 1)
            return c
        lax.fori_loop(0, NWINC // 2, _outer, 0)
        step(jnp.int32(NWINC - 1), 0)
        idx_wait(1)
        pltpu.make_async_copy(ones_b, sh_cnt.at[idx_d[0]], ssc[0]).wait()
        plsc.subcore_barrier()

        iota_f = lax.iota(jnp.int32, 16).astype(jnp.float32)
        onehots = [jnp.maximum(1.0 - jnp.abs(iota_f - float(r)), 0.0)
                   for r in range(16)]
        for j in range(RPT // WC):
            pltpu.sync_copy(sh_cnt.at[pl.ds(r0 + j * WC, WC)], ones_b)
            for q in range(WC // 16):
                acc = ones_b[q * 16, pl.ds(0, 16)] * onehots[0]
                for r in range(1, 16):
                    acc = acc + ones_b[q * 16 + r, pl.ds(0, 16)] * onehots[r]
                cw[j * (WC // 16) + q, :] = acc
        pltpu.sync_copy(cw, cnt_out.at[pl.ds(sid * (RPT // 16), RPT // 16)])


@functools.partial(
    pl.kernel,
    mesh=plsc.VectorSubcoreMesh(core_axis_name="c", subcore_axis_name="s",
                                num_cores=1),
    out_type=[jax.ShapeDtypeStruct((NPAD // 16, 16), jnp.float32)],
    scratch_types=[
        [pltpu.VMEM((WC,), jnp.int32)] * 2,
        pltpu.VMEM((WC, D), jnp.float32),
        pltpu.VMEM((RPT // 16, 16), jnp.float32),
        pltpu.VMEM_SHARED((NPAD, D), jnp.float32),
        [pltpu.SemaphoreType.DMA] * 2,
        [pltpu.SemaphoreType.DMA] * 2,
    ],
)
def _cnt_sc(dst_hbm, cnt_out, idx_d, ones_b, cw, sh_cnt, sic, ssc):
    _cnt_body(dst_hbm, cnt_out, idx_d, ones_b, cw, sh_cnt, sic, ssc)


def _update_body(h_ref, a0_ref, c0_ref, wt_ref, wb_ref,
                 bu_ref, lg_ref, lb_ref, g_ref, b_ref, o_ref):
    h = h_ref[...]
    cnt = c0_ref[...]
    s = a0_ref[...].astype(jnp.float32)
    am = (lg_ref[...] * s + lb_ref[...] * cnt) / jnp.clip(cnt, 1.0, None)
    u = (jnp.dot(h, wt_ref[...], preferred_element_type=jnp.float32)
         + jnp.dot(am, wb_ref[...], preferred_element_type=jnp.float32)
         + bu_ref[...])
    mu = jnp.mean(u, axis=-1, keepdims=True)
    var = jnp.mean(u * u, axis=-1, keepdims=True) - mu * mu
    y = (u - mu) * lax.rsqrt(var + 1e-5) * g_ref[...] + b_ref[...]
    o_ref[...] = jnp.maximum(y, 0.0) + h


def _update(h, aggr, cnt, p_conv, ln):
    return pl.pallas_call(
        _update_body,
        grid=(N // ROWS,),
        in_specs=[_rows(D),
                  pl.BlockSpec((ROWS, D), lambda i: (i, 0)),
                  pl.BlockSpec((ROWS, 1), lambda i: (i, 0)),
                  _full((D, D)), _full((D, D)), _full((1, D)),
                  _full((1, D)), _full((1, D)),
                  _full((1, D)), _full((1, D))],
        out_specs=_rows(D),
        out_shape=jax.ShapeDtypeStruct((N, D), jnp.float32),
    )(h, aggr, cnt,
      p_conv["upd"]["W"][:D], p_conv["upd"]["W"][D:],
      p_conv["upd"]["b"].reshape(1, D),
      p_conv["ln_g"].reshape(1, D), p_conv["ln_b"].reshape(1, D),
      ln["g"].reshape(1, D), ln["b"].reshape(1, D))


def _head_body(x_ref, w1_ref, b1_ref, w2_ref, b2_ref, o_ref):
    z = jnp.maximum(
        jnp.dot(x_ref[...], w1_ref[...], preferred_element_type=jnp.float32)
        + b1_ref[...], 0.0)
    o_ref[...] = (jnp.dot(z, w2_ref[...], preferred_element_type=jnp.float32)
                  + b2_ref[...])


def _head(x, p1, p2):
    return pl.pallas_call(
        _head_body,
        grid=(N // ROWS,),
        in_specs=[_rows(D), _full((D, D // 2)), _full((1, D // 2)),
                  _full((D // 2, 1)), _full((1, 1))],
        out_specs=pl.BlockSpec((ROWS, 1), lambda i: (i, 0)),
        out_shape=jax.ShapeDtypeStruct((N, 1), jnp.float32),
    )(x, p1["W"], p1["b"].reshape(1, -1), p2["W"], p2["b"].reshape(1, 1))


def kernel(x_drivers, x_races, params, ei_dr_ra, ei_ra_dr):
    ei_dr_ra = ei_dr_ra.astype(jnp.int32)
    ei_ra_dr = ei_ra_dr.astype(jnp.int32)
    enc = params["enc"]
    h_dr = _linrelu(x_drivers, enc["drivers"]["W"], enc["drivers"]["b"])
    h_ra = _linrelu(x_races, enc["races"]["W"], enc["races"]["b"])
    cnt_ra, = _cnt_sc(ei_dr_ra[1])
    cnt_dr, = _cnt_sc(ei_ra_dr[1])
    cnt_ra = cnt_ra.reshape(NPAD, 1)
    cnt_dr = cnt_dr.reshape(NPAD, 1)
    for lp in params["layers"]:
        p_ra = lp["dr_ra"]
        p_dr = lp["ra_dr"]
        pred_ra = _mlp2(h_ra, p_ra["pred1"]["W"], p_ra["pred1"]["b"],
                        p_ra["pred2"]["W"], p_ra["pred2"]["b"])
        pred_dr = _mlp2(h_dr, p_dr["pred1"]["W"], p_dr["pred1"]["b"],
                        p_dr["pred2"]["W"], p_dr["pred2"]["b"])
        aggr_ra, = _edge_sc(h_dr, pred_ra, ei_dr_ra[0], ei_dr_ra[1])
        aggr_dr, = _edge_sc(h_ra, pred_dr, ei_ra_dr[0], ei_ra_dr[1])
        aggr_ra = _unpack_aggr(aggr_ra)
        aggr_dr = _unpack_aggr(aggr_dr)
        new_ra = _update(h_ra, aggr_ra, cnt_ra, p_ra, lp["ln"]["races"])
        new_dr = _update(h_dr, aggr_dr, cnt_dr, p_dr, lp["ln"]["drivers"])
        h_ra, h_dr = new_ra, new_dr
    return _head(h_dr, params["head1"], params["head2"])
